# baseline (speedup 1.0000x reference)
import functools

import jax
import jax.numpy as jnp
from jax import lax
from jax.experimental import pallas as pl
from jax.experimental.pallas import tpu as pltpu
from jax.experimental.pallas import tpu_sc as plsc

_LANES = 16


@functools.cache
def _build_kernel(B, S, D, V):
    info = plsc.get_sparse_core_info()
    NW = info.num_cores * info.num_subcores
    rows_total = B * S
    rows_per_w = rows_total // NW
    C = S
    n_chunks = rows_per_w // C

    mesh = plsc.VectorSubcoreMesh(core_axis_name="c", subcore_axis_name="s")

    @functools.partial(
        pl.kernel,
        out_type=jax.ShapeDtypeStruct((rows_total, D), jnp.float32),
        mesh=mesh,
        scratch_types=[
            pltpu.VMEM((C,), jnp.int32),
            pltpu.VMEM((C, D), jnp.float32),
            pltpu.VMEM((S, D), jnp.float32),
            pltpu.SemaphoreType.DMA,
        ],
        compiler_params=pltpu.CompilerParams(use_tc_tiling_on_sc=False),
    )
    def embed(idx_hbm, table_hbm, pos_hbm, out_hbm, idx_v, rows_v, pos_v, sem):
        wid = lax.axis_index("s") * info.num_cores + lax.axis_index("c")
        base = wid * rows_per_w
        pltpu.sync_copy(pos_hbm, pos_v)

        def chunk_body(g, carry):
            row0 = base + g * C
            pltpu.sync_copy(idx_hbm.at[pl.ds(row0, C)], idx_v)
            pltpu.async_copy(table_hbm.at[idx_v], rows_v, sem).wait()

            def add_body(r, carry2):
                for d in range(D // _LANES):
                    x = pos_v[r, pl.ds(d * _LANES, _LANES)]
                    plsc.addupdate(rows_v.at[r, pl.ds(d * _LANES, _LANES)], x)
                return carry2

            lax.fori_loop(0, S, add_body, 0, unroll=2)
            pltpu.sync_copy(rows_v, out_hbm.at[pl.ds(row0, C)])
            return carry

        lax.fori_loop(0, n_chunks, chunk_body, 0)

    return embed


def kernel(inputs, token_table, position_table):
    B, S = inputs.shape
    V, D = token_table.shape
    idx_flat = inputs.reshape(B * S).astype(jnp.int32)
    fn = _build_kernel(B, S, D, V)
    out = fn(idx_flat, token_table, position_table)
    return out.reshape(B, S, D)


# speedup vs baseline: 1.1667x; 1.1667x over previous
import functools

import jax
import jax.numpy as jnp
from jax import lax
from jax.experimental import pallas as pl
from jax.experimental.pallas import tpu as pltpu
from jax.experimental.pallas import tpu_sc as plsc

_LANES = 16
_NBUF = 4


@functools.cache
def _build_kernel(B, S, D, V):
    info = plsc.get_sparse_core_info()
    NW = info.num_cores * info.num_subcores
    rows_total = B * S
    rows_per_w = rows_total // NW
    C = S
    n_chunks = rows_per_w // C
    assert n_chunks % _NBUF == 0

    mesh = plsc.VectorSubcoreMesh(core_axis_name="c", subcore_axis_name="s")

    @functools.partial(
        pl.kernel,
        out_type=jax.ShapeDtypeStruct((rows_total, D), jnp.float32),
        mesh=mesh,
        scratch_types=(
            [pltpu.VMEM((rows_per_w,), jnp.int32)]
            + [pltpu.VMEM((S, D), jnp.float32)]
            + [pltpu.VMEM((C, D), jnp.float32) for _ in range(_NBUF)]
            + [pltpu.SemaphoreType.DMA for _ in range(2 * _NBUF)]
        ),
        compiler_params=pltpu.CompilerParams(use_tc_tiling_on_sc=False),
    )
    def embed(idx_hbm, table_hbm, pos_hbm, out_hbm, idx_all, pos_v,
              *bufs_and_sems):
        rows = bufs_and_sems[:_NBUF]
        gsem = bufs_and_sems[_NBUF:2 * _NBUF]
        wsem = bufs_and_sems[2 * _NBUF:]
        wid = lax.axis_index("s") * info.num_cores + lax.axis_index("c")
        base = wid * rows_per_w

        pltpu.sync_copy(idx_hbm.at[pl.ds(base, rows_per_w)], idx_all)
        pltpu.sync_copy(pos_hbm, pos_v)

        def gather_desc(q, j):
            return pltpu.make_async_copy(
                table_hbm.at[idx_all.at[pl.ds(q * C, C)]],
                rows[j], gsem[j])

        def write_desc(q, j):
            return pltpu.make_async_copy(
                rows[j], out_hbm.at[pl.ds(base + q * C, C)], wsem[j])

        for j in range(_NBUF - 1):
            gather_desc(j, j).start()

        def block_body(blk, carry):
            for b in range(_NBUF):
                q = blk * _NBUF + b
                jprev = (b - 1) % _NBUF
                jnext = (b + _NBUF - 1) % _NBUF
                if b == 0:
                    @pl.when(blk >= 1)
                    def _():
                        write_desc(q - 1, jprev).wait()
                else:
                    write_desc(q - 1, jprev).wait()
                @pl.when(q + _NBUF - 1 < n_chunks)
                def _():
                    gather_desc(q + _NBUF - 1, jnext).start()
                gather_desc(q, b).wait()
                buf = rows[b]

                def add_body(r, carry2):
                    for d in range(D // _LANES):
                        x = pos_v[r, pl.ds(d * _LANES, _LANES)]
                        plsc.addupdate(buf.at[r, pl.ds(d * _LANES, _LANES)], x)
                    return carry2

                lax.fori_loop(0, S, add_body, 0, unroll=4)
                write_desc(q, b).start()
            return carry

        lax.fori_loop(0, n_chunks // _NBUF, block_body, 0)
        write_desc(n_chunks - 1, (n_chunks - 1) % _NBUF).wait()

    return embed


def kernel(inputs, token_table, position_table):
    B, S = inputs.shape
    V, D = token_table.shape
    idx_flat = inputs.reshape(B * S).astype(jnp.int32)
    fn = _build_kernel(B, S, D, V)
    out = fn(idx_flat, token_table, position_table)
    return out.reshape(B, S, D)


# speedup vs baseline: 1.1685x; 1.0016x over previous
import functools

import jax
import jax.numpy as jnp
from jax import lax
from jax.experimental import pallas as pl
from jax.experimental.pallas import tpu as pltpu
from jax.experimental.pallas import tpu_sc as plsc

_LANES = 16
_NBUF = 4


@functools.cache
def _build_kernel(B, S, D, V):
    info = plsc.get_sparse_core_info()
    NW = info.num_cores * info.num_subcores
    rows_total = B * S
    rows_per_w = rows_total // NW
    C = S
    n_chunks = rows_per_w // C
    assert n_chunks % _NBUF == 0

    mesh = plsc.VectorSubcoreMesh(core_axis_name="c", subcore_axis_name="s")

    @functools.partial(
        pl.kernel,
        out_type=jax.ShapeDtypeStruct((B, S, D), jnp.float32),
        mesh=mesh,
        scratch_types=(
            [pltpu.VMEM((rows_per_w,), jnp.int32)]
            + [pltpu.VMEM((S, D), jnp.float32)]
            + [pltpu.VMEM((C, D), jnp.float32) for _ in range(_NBUF)]
            + [pltpu.SemaphoreType.DMA for _ in range(2 * _NBUF)]
        ),
        compiler_params=pltpu.CompilerParams(use_tc_tiling_on_sc=False),
    )
    def embed(idx_hbm, table_hbm, pos_hbm, out_hbm, idx_all, pos_v,
              *bufs_and_sems):
        rows = bufs_and_sems[:_NBUF]
        gsem = bufs_and_sems[_NBUF:2 * _NBUF]
        wsem = bufs_and_sems[2 * _NBUF:]
        wid = lax.axis_index("s") * info.num_cores + lax.axis_index("c")
        base = wid * rows_per_w

        pltpu.sync_copy(idx_hbm.at[pl.ds(base, rows_per_w)], idx_all)
        pltpu.sync_copy(pos_hbm, pos_v)

        def gather_desc(q, j):
            return pltpu.make_async_copy(
                table_hbm.at[idx_all.at[pl.ds(q * C, C)]],
                rows[j], gsem[j])

        def write_desc(q, j):
            return pltpu.make_async_copy(
                rows[j], out_hbm.at[wid * n_chunks + q], wsem[j])

        for j in range(_NBUF - 1):
            gather_desc(j, j).start()

        def block_body(blk, carry):
            for b in range(_NBUF):
                q = blk * _NBUF + b
                jprev = (b - 1) % _NBUF
                jnext = (b + _NBUF - 1) % _NBUF
                if b == 0:
                    @pl.when(blk >= 1)
                    def _():
                        write_desc(q - 1, jprev).wait()
                else:
                    write_desc(q - 1, jprev).wait()
                @pl.when(q + _NBUF - 1 < n_chunks)
                def _():
                    gather_desc(q + _NBUF - 1, jnext).start()
                gather_desc(q, b).wait()
                buf = rows[b]

                def add_body(r, carry2):
                    for d in range(D // _LANES):
                        x = pos_v[r, pl.ds(d * _LANES, _LANES)]
                        plsc.addupdate(buf.at[r, pl.ds(d * _LANES, _LANES)], x)
                    return carry2

                lax.fori_loop(0, S, add_body, 0, unroll=4)
                write_desc(q, b).start()
            return carry

        lax.fori_loop(0, n_chunks // _NBUF, block_body, 0)
        write_desc(n_chunks - 1, (n_chunks - 1) % _NBUF).wait()

    return embed


def kernel(inputs, token_table, position_table):
    B, S = inputs.shape
    V, D = token_table.shape
    idx_flat = inputs.reshape(B * S).astype(jnp.int32)
    fn = _build_kernel(B, S, D, V)
    return fn(idx_flat, token_table, position_table)
